# baseline (speedup 1.0000x reference)
import functools

import jax
import jax.numpy as jnp
from jax import lax
from jax.experimental import pallas as pl
from jax.experimental.pallas import tpu as pltpu
from jax.experimental.pallas import tpu_sc as plsc


_TILE = 512
_RING = 4


def _sc_band_body(band_row0, ncols, rows_per, p_hbm, l_hbm, buf, in_sem,
                  out_sem):
    wid = lax.axis_index("s") * 2 + lax.axis_index("c")
    lbase = wid * rows_per

    def in_copy(k):
        r = band_row0 + lbase + k
        off = pl.multiple_of((r * (r + 1)) // 2, 128)
        return pltpu.make_async_copy(
            p_hbm.at[pl.ds(off, ncols)], buf.at[k % _RING], in_sem)

    def out_copy(k):
        return pltpu.make_async_copy(
            buf.at[k % _RING], l_hbm.at[lbase + k], out_sem)

    in_copy(0).start()

    def body(k, carry):
        @pl.when(k + 1 < rows_per)
        def _():
            @pl.when(k + 1 >= _RING)
            def _():
                out_copy(k + 1 - _RING).wait()

            in_copy(k + 1).start()

        in_copy(k).wait()
        out_copy(k).start()
        return carry

    lax.fori_loop(0, rows_per, body, 0)
    for _ in range(min(_RING, rows_per)):
        out_copy(0).wait()


def _sc_band_call(n, t, b, n_workers):
    ncols = (b + 1) * t
    return functools.partial(
        pl.kernel,
        mesh=plsc.VectorSubcoreMesh(core_axis_name="c", subcore_axis_name="s"),
        out_type=jax.ShapeDtypeStruct((t, ncols), jnp.float32),
        scratch_types=[
            pltpu.VMEM((_RING, ncols), jnp.float32),
            pltpu.SemaphoreType.DMA,
            pltpu.SemaphoreType.DMA,
        ],
    )(functools.partial(_sc_band_body, b * t, ncols, t // n_workers))


def _tc_band_body(t, b, has_prev, *refs):
    if has_prev:
        l_ref, a_ref, _prev, o_ref, s_ref = refs
    else:
        l_ref, a_ref, o_ref, s_ref = refs
    k = pl.program_id(0)
    side = pl.program_id(1)

    @pl.when(side == 0)
    def _():
        l = l_ref[...]

        @pl.when(k == b)
        def _():
            rows = lax.broadcasted_iota(jnp.int32, (t, t), 0)
            cols = lax.broadcasted_iota(jnp.int32, (t, t), 1)
            sym = jnp.where(cols <= rows, l, l.T)
            s_ref[...] = 1.0 / (1.0 + jnp.exp(-sym))

        @pl.when(k != b)
        def _():
            s_ref[...] = 1.0 / (1.0 + jnp.exp(-l))

        o_ref[...] = s_ref[...] * a_ref[...]

    @pl.when(side == 1)
    def _():
        o_ref[...] = s_ref[...].T * a_ref[...]


def _tc_band_call(n, t, b, has_prev):
    def sided_map(k, s):
        return (jnp.where(s == 0, b, k), jnp.where(s == 0, k, b))

    in_specs = [
        pl.BlockSpec((t, t), lambda k, s: (0, k)),
        pl.BlockSpec((t, t), sided_map),
    ]
    if has_prev:
        in_specs.append(pl.BlockSpec(memory_space=pl.ANY))

    return pl.pallas_call(
        functools.partial(_tc_band_body, t, b, has_prev),
        grid=(b + 1, 2),
        in_specs=in_specs,
        out_specs=pl.BlockSpec((t, t), sided_map),
        out_shape=jax.ShapeDtypeStruct((n, n), jnp.float32),
        scratch_shapes=[pltpu.VMEM((t, t), jnp.float32)],
        input_output_aliases={2: 0} if has_prev else {},
        compiler_params=pltpu.CompilerParams(
            dimension_semantics=("arbitrary", "arbitrary")),
    )


def kernel(P_vec, adj):
    n = adj.shape[0]
    t = min(_TILE, n)
    nb = n // t

    info = plsc.get_sparse_core_info()
    n_workers = info.num_cores * info.num_subcores

    bands = [_sc_band_call(n, t, b, n_workers)(P_vec) for b in range(nb)]

    out = _tc_band_call(n, t, nb - 1, False)(bands[nb - 1], adj)
    for b in range(nb - 2, -1, -1):
        out = _tc_band_call(n, t, b, True)(bands[b], adj, out)
    return out


# speedup vs baseline: 1.2143x; 1.2143x over previous
import functools

import jax
import jax.numpy as jnp
from jax import lax
from jax.experimental import pallas as pl
from jax.experimental.pallas import tpu as pltpu
from jax.experimental.pallas import tpu_sc as plsc


_TILE = 1024
_RING = 4


_QUARTERS = 4


def _sc_band_body(band_row0, ncols_base, t, n_workers, p_hbm, l_hbm, buf,
                  in_sem, out_sem):
    wid = lax.axis_index("s") * 2 + lax.axis_index("c")
    rows_per = t // n_workers
    lbase = wid * rows_per
    per_q = n_workers // _QUARTERS

    def run(ncols_g):
        def in_copy(k):
            r = band_row0 + lbase + k
            off = pl.multiple_of((r * (r + 1)) // 2, 128)
            return pltpu.make_async_copy(
                p_hbm.at[pl.ds(off, ncols_g)],
                buf.at[k % _RING, pl.ds(0, ncols_g)], in_sem)

        def out_copy(k):
            return pltpu.make_async_copy(
                buf.at[k % _RING, pl.ds(0, ncols_g)],
                l_hbm.at[lbase + k, pl.ds(0, ncols_g)], out_sem)

        in_copy(0).start()

        def body(k, carry):
            @pl.when(k + 1 < rows_per)
            def _():
                @pl.when(k + 1 >= _RING)
                def _():
                    out_copy(k + 1 - _RING).wait()

                in_copy(k + 1).start()

            in_copy(k).wait()
            out_copy(k).start()
            return carry

        lax.fori_loop(0, rows_per, body, 0)
        for _ in range(min(_RING, rows_per)):
            out_copy(0).wait()

    for g in range(_QUARTERS):
        @pl.when((wid >= g * per_q) & (wid < (g + 1) * per_q))
        def _(g=g):
            run(ncols_base + (g + 1) * (t // _QUARTERS))


def _sc_band_call(n, t, b, n_workers):
    ncols = (b + 1) * t
    return functools.partial(
        pl.kernel,
        mesh=plsc.VectorSubcoreMesh(core_axis_name="c", subcore_axis_name="s"),
        out_type=jax.ShapeDtypeStruct((t, ncols), jnp.float32),
        scratch_types=[
            pltpu.VMEM((_RING, ncols), jnp.float32),
            pltpu.SemaphoreType.DMA,
            pltpu.SemaphoreType.DMA,
        ],
    )(functools.partial(_sc_band_body, b * t, b * t, t, n_workers))


def _tc_band_body(t, b, has_prev, *refs):
    if has_prev:
        l_ref, a_ref, _prev, o_ref, s_ref = refs
    else:
        l_ref, a_ref, o_ref, s_ref = refs
    k = pl.program_id(0)
    side = pl.program_id(1)

    @pl.when(side == 0)
    def _():
        l = l_ref[...]

        @pl.when(k == b)
        def _():
            rows = lax.broadcasted_iota(jnp.int32, (t, t), 0)
            cols = lax.broadcasted_iota(jnp.int32, (t, t), 1)
            sym = jnp.where(cols <= rows, l, l.T)
            s_ref[...] = 1.0 / (1.0 + jnp.exp(-sym))

        @pl.when(k != b)
        def _():
            s_ref[...] = 1.0 / (1.0 + jnp.exp(-l))

        o_ref[...] = s_ref[...] * a_ref[...]

    @pl.when(side == 1)
    def _():
        o_ref[...] = s_ref[...].T * a_ref[...]


def _tc_band_call(n, t, b, has_prev):
    def sided_map(k, s):
        return (jnp.where(s == 0, b, k), jnp.where(s == 0, k, b))

    in_specs = [
        pl.BlockSpec((t, t), lambda k, s: (0, k)),
        pl.BlockSpec((t, t), sided_map),
    ]
    if has_prev:
        in_specs.append(pl.BlockSpec(memory_space=pl.ANY))

    return pl.pallas_call(
        functools.partial(_tc_band_body, t, b, has_prev),
        grid=(b + 1, 2),
        in_specs=in_specs,
        out_specs=pl.BlockSpec((t, t), sided_map),
        out_shape=jax.ShapeDtypeStruct((n, n), jnp.float32),
        scratch_shapes=[pltpu.VMEM((t, t), jnp.float32)],
        input_output_aliases={2: 0} if has_prev else {},
        compiler_params=pltpu.CompilerParams(
            dimension_semantics=("arbitrary", "arbitrary")),
    )


def kernel(P_vec, adj):
    n = adj.shape[0]
    t = min(_TILE, n)
    nb = n // t

    info = plsc.get_sparse_core_info()
    n_workers = info.num_cores * info.num_subcores

    bands = [_sc_band_call(n, t, b, n_workers)(P_vec) for b in range(nb)]

    out = _tc_band_call(n, t, nb - 1, False)(bands[nb - 1], adj)
    for b in range(nb - 2, -1, -1):
        out = _tc_band_call(n, t, b, True)(bands[b], adj, out)
    return out
